# baseline (speedup 1.0000x reference)
import functools
import jax
import jax.numpy as jnp
from jax import lax
from jax.experimental import pallas as pl
from jax.experimental.pallas import tpu as pltpu
from jax.experimental.pallas import tpu_sc as plsc

_NC = 2
_NS = 16
_T = 200
_B = 16384
_BW = _B // (_NC * _NS)
_TT = _T // 8
_BT = _BW // 128


def _sc_body(idxT_hbm, out_hbm, idx_v, buf):
    wid = lax.axis_index("s") * _NC + lax.axis_index("c")
    b0w = wid * _BW

    zeros16f = jnp.zeros((16,), jnp.float32)
    ones16f = jnp.ones((16,), jnp.float32)
    iota16 = lax.iota(jnp.int32, 16)

    def zrow(r, carry):
        for tj in range(8):
            for cj in range(8):
                buf[r, tj, pl.ds(cj * 16, 16)] = zeros16f
        return carry

    lax.fori_loop(0, 100, zrow, 0)

    def unit(u, carry):
        tt = u // _BT
        bb = u % _BT
        t0 = tt * 8
        b0 = b0w + bb * 128
        pltpu.sync_copy(idxT_hbm.at[pl.ds(t0, 8), pl.ds(b0, 128)], idx_v)
        for tj in range(8):
            trow = jnp.full((16,), tj, jnp.int32)
            for cj in range(8):
                v = idx_v[tj, pl.ds(cj * 16, 16)]
                plsc.store_scatter(
                    buf, [v - 1, trow, cj * 16 + iota16], ones16f,
                    mask=v > 0)
        pltpu.sync_copy(buf, out_hbm.at[:, pl.ds(t0, 8), pl.ds(b0, 128)])
        for tj in range(8):
            trow = jnp.full((16,), tj, jnp.int32)
            for cj in range(8):
                v = idx_v[tj, pl.ds(cj * 16, 16)]
                plsc.store_scatter(
                    buf, [v - 1, trow, cj * 16 + iota16], zeros16f,
                    mask=v > 0)
        return carry

    lax.fori_loop(0, _TT * _BT, unit, 0)


def kernel(inputs, z_weights):
    del z_weights
    B, T = inputs.shape
    idx_t = inputs.astype(jnp.int32).T
    mesh = plsc.VectorSubcoreMesh(core_axis_name="c", subcore_axis_name="s")
    k = functools.partial(
        pl.kernel,
        out_type=jax.ShapeDtypeStruct((100, T, B), jnp.float32),
        mesh=mesh,
        scratch_types=[
            pltpu.VMEM((8, 128), jnp.int32),
            pltpu.VMEM((100, 8, 128), jnp.float32),
        ],
        compiler_params=pltpu.CompilerParams(needs_layout_passes=False),
    )(_sc_body)
    out_t = k(idx_t)
    return out_t.transpose(2, 1, 0)


# speedup vs baseline: 1.1751x; 1.1751x over previous
import functools
import jax
import jax.numpy as jnp
from jax import lax
from jax.experimental import pallas as pl
from jax.experimental.pallas import tpu as pltpu
from jax.experimental.pallas import tpu_sc as plsc

_NC = 2
_NS = 16
_T = 200
_B = 16384
_BW = _B // (_NC * _NS)
_TT = _T // 8
_BT = _BW // 128
_NU = _TT * _BT * 2


def _scatter_unit(buf, idx_v, val16, p):
    iota16 = lax.iota(jnp.int32, 16)
    lo = 50 * p
    hi = 50 * (p + 1)
    for tj in range(8):
        trow = jnp.full((16,), tj, jnp.int32)
        for cj in range(8):
            v = idx_v[tj, pl.ds(cj * 16, 16)]
            m = (v > lo) & (v <= hi)
            plsc.store_scatter(
                buf, [v - 1 - lo, trow, cj * 16 + iota16], val16, mask=m)


def _sc_body(idxT_hbm, out_hbm, idx_v, bufs, sems):
    wid = lax.axis_index("s") * _NC + lax.axis_index("c")
    b0w = wid * _BW

    zeros16f = jnp.zeros((16,), jnp.float32)
    ones16f = jnp.ones((16,), jnp.float32)

    def zrow(r, carry):
        for s in range(2):
            for tj in range(8):
                for cj in range(8):
                    bufs[s, r, tj, pl.ds(cj * 16, 16)] = zeros16f
        return carry

    lax.fori_loop(0, 50, zrow, 0)

    def dst(u):
        p = lax.rem(u, 2)
        tb = u // 2
        tt = tb // _BT
        bb = lax.rem(tb, _BT)
        return out_hbm.at[pl.ds(p * 50, 50), pl.ds(tt * 8, 8),
                          pl.ds(b0w + bb * 128, 128)]

    def unit(u, carry):
        p = lax.rem(u, 2)

        @pl.when(u >= 2)
        def _():
            pltpu.make_async_copy(bufs.at[p], dst(u - 2), sems.at[p]).wait()
            _scatter_unit(bufs.at[p], idx_v.at[p], zeros16f, p)

        tb = u // 2
        tt = tb // _BT
        bb = lax.rem(tb, _BT)
        pltpu.sync_copy(
            idxT_hbm.at[pl.ds(tt * 8, 8), pl.ds(b0w + bb * 128, 128)],
            idx_v.at[p])
        _scatter_unit(bufs.at[p], idx_v.at[p], ones16f, p)
        pltpu.make_async_copy(bufs.at[p], dst(u), sems.at[p]).start()
        return carry

    lax.fori_loop(0, _NU, unit, 0)

    for k in range(2):
        u = _NU - 2 + k
        pltpu.make_async_copy(
            bufs.at[u % 2], dst(jnp.int32(u)), sems.at[u % 2]).wait()


def kernel(inputs, z_weights):
    del z_weights
    B, T = inputs.shape
    idx_t = inputs.astype(jnp.int32).T
    mesh = plsc.VectorSubcoreMesh(core_axis_name="c", subcore_axis_name="s")
    k = functools.partial(
        pl.kernel,
        out_type=jax.ShapeDtypeStruct((100, T, B), jnp.float32),
        mesh=mesh,
        scratch_types=[
            pltpu.VMEM((2, 8, 128), jnp.int32),
            pltpu.VMEM((2, 50, 8, 128), jnp.float32),
            pltpu.SemaphoreType.DMA((2,)),
        ],
        compiler_params=pltpu.CompilerParams(needs_layout_passes=False),
    )(_sc_body)
    out_t = k(idx_t)
    return out_t.transpose(2, 1, 0)


# speedup vs baseline: 1.2252x; 1.0427x over previous
import functools
import jax
import jax.numpy as jnp
from jax import lax
from jax.experimental import pallas as pl
from jax.experimental.pallas import tpu as pltpu
from jax.experimental.pallas import tpu_sc as plsc

_NC = 2
_NS = 16
_T = 200
_B = 16384
_BW = _B // (_NC * _NS)
_TT = _T // 8
_BT = _BW // 128
_NU = _TT * _BT * 2


def _scatter_unit(buf, idx_v, bb, val16, p):
    iota16 = lax.iota(jnp.int32, 16)
    lo = 50 * p
    hi = 50 * (p + 1)
    for tj in range(8):
        trow = jnp.full((16,), tj, jnp.int32)
        for cj in range(8):
            v = idx_v[tj, pl.ds(bb * 128 + cj * 16, 16)]
            m = (v > lo) & (v <= hi)
            plsc.store_scatter(
                buf, [v - 1 - lo, trow, cj * 16 + iota16], val16, mask=m)


def _sc_body(idxT_hbm, out_hbm, idx_v, bufs, sems):
    wid = lax.axis_index("s") * _NC + lax.axis_index("c")
    b0w = wid * _BW

    zeros16f = jnp.zeros((16,), jnp.float32)
    ones16f = jnp.ones((16,), jnp.float32)

    def zrow(r, carry):
        for s in range(2):
            for tj in range(8):
                for cj in range(8):
                    bufs[s, r, tj, pl.ds(cj * 16, 16)] = zeros16f
        return carry

    lax.fori_loop(0, 50, zrow, 0)

    def dst(u):
        p = lax.rem(u, 2)
        tb = u // 2
        tt = tb // _BT
        bb = lax.rem(tb, _BT)
        return out_hbm.at[pl.ds(p * 50, 50), pl.ds(tt * 8, 8),
                          pl.ds(b0w + bb * 128, 128)]

    def unit(u, carry):
        p = lax.rem(u, 2)
        tb = u // 2
        tt = tb // _BT
        bb = lax.rem(tb, _BT)

        @pl.when(u >= 2)
        def _():
            tb2 = (u - 2) // 2
            g2 = lax.rem(tb2 // _BT, 2)
            bb2 = lax.rem(tb2, _BT)
            pltpu.make_async_copy(bufs.at[p], dst(u - 2), sems.at[p]).wait()
            _scatter_unit(bufs.at[p], idx_v.at[g2], bb2, zeros16f, p)

        @pl.when(lax.rem(u, 2 * _BT) == 0)
        def _():
            pltpu.sync_copy(
                idxT_hbm.at[pl.ds(tt * 8, 8), pl.ds(b0w, _BW)],
                idx_v.at[lax.rem(tt, 2)])

        _scatter_unit(bufs.at[p], idx_v.at[lax.rem(tt, 2)], bb, ones16f, p)
        pltpu.make_async_copy(bufs.at[p], dst(u), sems.at[p]).start()
        return carry

    lax.fori_loop(0, _NU, unit, 0)

    for k in range(2):
        u = _NU - 2 + k
        pltpu.make_async_copy(
            bufs.at[u % 2], dst(jnp.int32(u)), sems.at[u % 2]).wait()


def kernel(inputs, z_weights):
    del z_weights
    B, T = inputs.shape
    idx_t = inputs.astype(jnp.int32).T
    mesh = plsc.VectorSubcoreMesh(core_axis_name="c", subcore_axis_name="s")
    k = functools.partial(
        pl.kernel,
        out_type=jax.ShapeDtypeStruct((100, T, B), jnp.float32),
        mesh=mesh,
        scratch_types=[
            pltpu.VMEM((2, 8, _BW), jnp.int32),
            pltpu.VMEM((2, 50, 8, 128), jnp.float32),
            pltpu.SemaphoreType.DMA((2,)),
        ],
        compiler_params=pltpu.CompilerParams(needs_layout_passes=False),
    )(_sc_body)
    out_t = k(idx_t)
    return out_t.transpose(2, 1, 0)


# speedup vs baseline: 1.2313x; 1.0049x over previous
import functools
import jax
import jax.numpy as jnp
from jax import lax
from jax.experimental import pallas as pl
from jax.experimental.pallas import tpu as pltpu
from jax.experimental.pallas import tpu_sc as plsc

_NC = 2
_NS = 16
_T = 200
_B = 16384
_BW = _B // (_NC * _NS)
_TT = _T // 8
_BT = _BW // 128
_NU = _TT * _BT * 2


def _scatter_unit(buf, idx_v, bb, val16, p):
    iota16 = lax.iota(jnp.int32, 16)
    lo = 50 * p
    hi = 50 * (p + 1)
    for tj in range(8):
        trow = jnp.full((16,), tj, jnp.int32)
        for cj in range(8):
            v = idx_v[tj, pl.ds(bb * 128 + cj * 16, 16)]
            m = (v > lo) & (v <= hi)
            plsc.store_scatter(
                buf, [v - 1 - lo, trow, cj * 16 + iota16], val16, mask=m)


def _sc_body(idxT_hbm, out_hbm, idx_v, bufs, sems, idx_sem):
    wid = lax.axis_index("s") * _NC + lax.axis_index("c")
    b0w = wid * _BW

    def idx_copy(tt, g):
        return pltpu.make_async_copy(
            idxT_hbm.at[pl.ds(tt * 8, 8), pl.ds(b0w, _BW)],
            idx_v.at[g], idx_sem)

    zeros16f = jnp.zeros((16,), jnp.float32)
    ones16f = jnp.ones((16,), jnp.float32)

    def zrow(r, carry):
        for s in range(2):
            for tj in range(8):
                for cj in range(8):
                    bufs[s, r, tj, pl.ds(cj * 16, 16)] = zeros16f
        return carry

    lax.fori_loop(0, 50, zrow, 0)

    idx_copy(0, 0).start()
    idx_copy(0, 0).wait()

    def dst(u):
        p = lax.rem(u, 2)
        tb = u // 2
        tt = tb // _BT
        bb = lax.rem(tb, _BT)
        return out_hbm.at[pl.ds(p * 50, 50), pl.ds(tt * 8, 8),
                          pl.ds(b0w + bb * 128, 128)]

    def unit(u, carry):
        p = lax.rem(u, 2)
        tb = u // 2
        tt = tb // _BT
        bb = lax.rem(tb, _BT)

        @pl.when(u >= 2)
        def _():
            tb2 = (u - 2) // 2
            g2 = lax.rem(tb2 // _BT, 2)
            bb2 = lax.rem(tb2, _BT)
            pltpu.make_async_copy(bufs.at[p], dst(u - 2), sems.at[p]).wait()
            _scatter_unit(bufs.at[p], idx_v.at[g2], bb2, zeros16f, p)

        @pl.when((lax.rem(u, 2 * _BT) == 0) & (u > 0))
        def _():
            idx_copy(tt, lax.rem(tt, 2)).wait()

        @pl.when((lax.rem(u, 2 * _BT) == 2) & (u < _NU - 2 * _BT))
        def _():
            idx_copy(tt + 1, lax.rem(tt + 1, 2)).start()

        _scatter_unit(bufs.at[p], idx_v.at[lax.rem(tt, 2)], bb, ones16f, p)
        pltpu.make_async_copy(bufs.at[p], dst(u), sems.at[p]).start()
        return carry

    lax.fori_loop(0, _NU, unit, 0)

    for k in range(2):
        u = _NU - 2 + k
        pltpu.make_async_copy(
            bufs.at[u % 2], dst(jnp.int32(u)), sems.at[u % 2]).wait()


def kernel(inputs, z_weights):
    del z_weights
    B, T = inputs.shape
    idx_t = inputs.astype(jnp.int32).T
    mesh = plsc.VectorSubcoreMesh(core_axis_name="c", subcore_axis_name="s")
    k = functools.partial(
        pl.kernel,
        out_type=jax.ShapeDtypeStruct((100, T, B), jnp.float32),
        mesh=mesh,
        scratch_types=[
            pltpu.VMEM((2, 8, _BW), jnp.int32),
            pltpu.VMEM((2, 50, 8, 128), jnp.float32),
            pltpu.SemaphoreType.DMA((2,)),
            pltpu.SemaphoreType.DMA,
        ],
        compiler_params=pltpu.CompilerParams(needs_layout_passes=False),
    )(_sc_body)
    out_t = k(idx_t)
    return out_t.transpose(2, 1, 0)
